# baseline (speedup 1.0000x reference)
import jax
import jax.numpy as jnp
from jax.experimental import pallas as pl
from jax.experimental.pallas import tpu as pltpu

T = 500
DROPOUT_P = 0.1
TPAD = 512
RR = 512


def _ddpm_body(x_ref, n_ref, ts_ref, u_ref, ta_ref, tb_ref,
               xt_ref, nout_ref, tn_ref, cm_ref, a_ref, b_ref):
    i = pl.program_id(0)

    @pl.when(i == 0)
    def _():
        ts = ts_ref[...]
        row = jax.lax.broadcasted_iota(jnp.int32, (TPAD, ts.shape[1]), 0)
        onehot = row == ts
        a_ref[...] = jnp.sum(jnp.where(onehot, ta_ref[...], 0.0),
                             axis=0, keepdims=True)
        b_ref[...] = jnp.sum(jnp.where(onehot, tb_ref[...], 0.0),
                             axis=0, keepdims=True)
        tn_ref[...] = ts.astype(jnp.float32) / T
        cm_ref[...] = (u_ref[...] < DROPOUT_P).astype(jnp.float32)

    n = n_ref[...]
    xt_ref[...] = a_ref[...] * x_ref[...] + b_ref[...] * n
    nout_ref[...] = n


def kernel(x, cls, timestep, noise, u, sqrt_abar_t, sqrt_abar_t1):
    B, C, H, W = x.shape
    F = C * H * W
    xv = x.transpose(1, 2, 3, 0).reshape(F, B)
    nv = noise.transpose(1, 2, 3, 0).reshape(F, B)
    ts2 = timestep.reshape(1, B)
    u2 = u.reshape(1, B)
    ta = jnp.zeros((TPAD, 1), jnp.float32).at[:T, 0].set(sqrt_abar_t)
    tb = jnp.zeros((TPAD, 1), jnp.float32).at[:T, 0].set(sqrt_abar_t1)

    grid = (F // RR,)
    big = pl.BlockSpec((RR, B), lambda i: (i, 0))
    vec = pl.BlockSpec((1, B), lambda i: (0, 0))
    tab = pl.BlockSpec((TPAD, 1), lambda i: (0, 0))

    xt, nout, tn2, cm2 = pl.pallas_call(
        _ddpm_body,
        grid=grid,
        in_specs=[big, big, vec, vec, tab, tab],
        out_specs=[big, big, vec, vec],
        out_shape=[
            jax.ShapeDtypeStruct((F, B), jnp.float32),
            jax.ShapeDtypeStruct((F, B), jnp.float32),
            jax.ShapeDtypeStruct((1, B), jnp.float32),
            jax.ShapeDtypeStruct((1, B), jnp.float32),
        ],
        scratch_shapes=[pltpu.VMEM((1, B), jnp.float32),
                        pltpu.VMEM((1, B), jnp.float32)],
    )(xv, nv, ts2, u2, ta, tb)

    img = lambda v: v.reshape(C, H, W, B).transpose(3, 0, 1, 2)
    return (img(nout), img(xt), cls, tn2.reshape(B), cm2.reshape(B))


# speedup vs baseline: 1.1331x; 1.1331x over previous
import jax
import jax.numpy as jnp
from jax.experimental import pallas as pl
from jax.experimental.pallas import tpu as pltpu

T = 500
DROPOUT_P = 0.1
TPAD = 512
RR = 2048


def _ddpm_body(x_ref, n_ref, ts_ref, u_ref, ta_ref, tb_ref,
               xt_ref, nout_ref, tn_ref, cm_ref, a_ref, b_ref):
    i = pl.program_id(0)

    @pl.when(i == 0)
    def _():
        ts = ts_ref[...]
        row = jax.lax.broadcasted_iota(jnp.int32, (TPAD, ts.shape[1]), 0)
        onehot = row == ts
        a_ref[...] = jnp.sum(jnp.where(onehot, ta_ref[...], 0.0),
                             axis=0, keepdims=True)
        b_ref[...] = jnp.sum(jnp.where(onehot, tb_ref[...], 0.0),
                             axis=0, keepdims=True)
        tn_ref[...] = ts.astype(jnp.float32) / T
        cm_ref[...] = (u_ref[...] < DROPOUT_P).astype(jnp.float32)

    n = n_ref[...]
    xt_ref[...] = a_ref[...] * x_ref[...] + b_ref[...] * n
    nout_ref[...] = n


def kernel(x, cls, timestep, noise, u, sqrt_abar_t, sqrt_abar_t1):
    B, C, H, W = x.shape
    F = C * H * W
    xv = x.transpose(1, 2, 3, 0).reshape(F, B)
    nv = noise.transpose(1, 2, 3, 0).reshape(F, B)
    ts2 = timestep.reshape(1, B)
    u2 = u.reshape(1, B)
    ta = jnp.zeros((TPAD, 1), jnp.float32).at[:T, 0].set(sqrt_abar_t)
    tb = jnp.zeros((TPAD, 1), jnp.float32).at[:T, 0].set(sqrt_abar_t1)

    grid = (F // RR,)
    big = pl.BlockSpec((RR, B), lambda i: (i, 0))
    vec = pl.BlockSpec((1, B), lambda i: (0, 0))
    tab = pl.BlockSpec((TPAD, 1), lambda i: (0, 0))

    xt, nout, tn2, cm2 = pl.pallas_call(
        _ddpm_body,
        grid=grid,
        in_specs=[big, big, vec, vec, tab, tab],
        out_specs=[big, big, vec, vec],
        out_shape=[
            jax.ShapeDtypeStruct((F, B), jnp.float32),
            jax.ShapeDtypeStruct((F, B), jnp.float32),
            jax.ShapeDtypeStruct((1, B), jnp.float32),
            jax.ShapeDtypeStruct((1, B), jnp.float32),
        ],
        scratch_shapes=[pltpu.VMEM((1, B), jnp.float32),
                        pltpu.VMEM((1, B), jnp.float32)],
    )(xv, nv, ts2, u2, ta, tb)

    img = lambda v: v.reshape(C, H, W, B).transpose(3, 0, 1, 2)
    return (img(nout), img(xt), cls, tn2.reshape(B), cm2.reshape(B))


# speedup vs baseline: 1.1745x; 1.0366x over previous
import jax
import jax.numpy as jnp
from jax.experimental import pallas as pl
from jax.experimental.pallas import tpu as pltpu

T = 500
DROPOUT_P = 0.1
TPAD = 512
RR = 3072


def _ddpm_body(x_ref, n_ref, ts_ref, u_ref, ta_ref, tb_ref,
               xt_ref, nout_ref, tn_ref, cm_ref, a_ref, b_ref):
    i = pl.program_id(0)

    @pl.when(i == 0)
    def _():
        ts = ts_ref[...]
        row = jax.lax.broadcasted_iota(jnp.int32, (TPAD, ts.shape[1]), 0)
        onehot = row == ts
        a_ref[...] = jnp.sum(jnp.where(onehot, ta_ref[...], 0.0),
                             axis=0, keepdims=True)
        b_ref[...] = jnp.sum(jnp.where(onehot, tb_ref[...], 0.0),
                             axis=0, keepdims=True)
        tn_ref[...] = ts.astype(jnp.float32) / T
        cm_ref[...] = (u_ref[...] < DROPOUT_P).astype(jnp.float32)

    n = n_ref[...]
    xt_ref[...] = a_ref[...] * x_ref[...] + b_ref[...] * n
    nout_ref[...] = n


def kernel(x, cls, timestep, noise, u, sqrt_abar_t, sqrt_abar_t1):
    B, C, H, W = x.shape
    F = C * H * W
    xv = x.transpose(1, 2, 3, 0).reshape(F, B)
    nv = noise.transpose(1, 2, 3, 0).reshape(F, B)
    ts2 = timestep.reshape(1, B)
    u2 = u.reshape(1, B)
    ta = jnp.zeros((TPAD, 1), jnp.float32).at[:T, 0].set(sqrt_abar_t)
    tb = jnp.zeros((TPAD, 1), jnp.float32).at[:T, 0].set(sqrt_abar_t1)

    grid = (F // RR,)
    big = pl.BlockSpec((RR, B), lambda i: (i, 0))
    vec = pl.BlockSpec((1, B), lambda i: (0, 0))
    tab = pl.BlockSpec((TPAD, 1), lambda i: (0, 0))

    xt, nout, tn2, cm2 = pl.pallas_call(
        _ddpm_body,
        grid=grid,
        in_specs=[big, big, vec, vec, tab, tab],
        out_specs=[big, big, vec, vec],
        out_shape=[
            jax.ShapeDtypeStruct((F, B), jnp.float32),
            jax.ShapeDtypeStruct((F, B), jnp.float32),
            jax.ShapeDtypeStruct((1, B), jnp.float32),
            jax.ShapeDtypeStruct((1, B), jnp.float32),
        ],
        scratch_shapes=[pltpu.VMEM((1, B), jnp.float32),
                        pltpu.VMEM((1, B), jnp.float32)],
    )(xv, nv, ts2, u2, ta, tb)

    img = lambda v: v.reshape(C, H, W, B).transpose(3, 0, 1, 2)
    return (img(nout), img(xt), cls, tn2.reshape(B), cm2.reshape(B))
